# initial kernel scaffold (unmeasured)
import jax
import jax.numpy as jnp
from jax import lax
from jax.experimental import pallas as pl
from jax.experimental.pallas import tpu as pltpu

N_DEV = 32


def kernel(x, w_mat):
    m, k_per = x.shape
    _, n = w_mat.shape
    chunk_rows = m // N_DEV

    def body(x_ref, w_ref, out_ref, comm_ref, send_sems, recv_sems, credit_sem):
        my = lax.axis_index("i")
        left = jnp.mod(my - 1, N_DEV)
        right = jnp.mod(my + 1, N_DEV)

        barrier_sem = pltpu.get_barrier_semaphore()
        pl.semaphore_signal(barrier_sem, inc=1, device_id=(left,),
                            device_id_type=pl.DeviceIdType.MESH)
        pl.semaphore_signal(barrier_sem, inc=1, device_id=(right,),
                            device_id_type=pl.DeviceIdType.MESH)
        pl.semaphore_wait(barrier_sem, 2)

        out_ref[:, :] = jnp.dot(x_ref[:, :], w_ref[:, :],
                                preferred_element_type=jnp.float32)

        def out_chunk(idx):
            return out_ref.at[pl.ds(idx * chunk_rows, chunk_rows), :]

        def rs_step(s, carry):
            slot = jnp.mod(s, 2)
            send_idx = jnp.mod(my - s, N_DEV)
            recv_idx = jnp.mod(my - s - 1, N_DEV)

            @pl.when(s > 0)
            def _():
                pl.semaphore_wait(credit_sem, 1)

            rdma = pltpu.make_async_remote_copy(
                src_ref=out_chunk(send_idx),
                dst_ref=comm_ref.at[slot],
                send_sem=send_sems.at[slot],
                recv_sem=recv_sems.at[slot],
                device_id=(right,),
                device_id_type=pl.DeviceIdType.MESH,
            )
            rdma.start()
            rdma.wait()

            out_ref[pl.ds(recv_idx * chunk_rows, chunk_rows), :] += comm_ref[slot]
            pl.semaphore_signal(credit_sem, inc=1, device_id=(left,),
                                device_id_type=pl.DeviceIdType.MESH)
            return carry

        lax.fori_loop(0, N_DEV - 1, rs_step, 0)

        def ag_step(t, carry):
            slot = jnp.mod(t + 1, 2)
            send_idx = jnp.mod(my + 1 - t, N_DEV)

            pl.semaphore_wait(credit_sem, 1)

            rdma = pltpu.make_async_remote_copy(
                src_ref=out_chunk(send_idx),
                dst_ref=out_chunk(send_idx),
                send_sem=send_sems.at[slot],
                recv_sem=recv_sems.at[slot],
                device_id=(right,),
                device_id_type=pl.DeviceIdType.MESH,
            )
            rdma.start()
            rdma.wait()

            @pl.when(t < N_DEV - 2)
            def _():
                pl.semaphore_signal(credit_sem, inc=1, device_id=(left,),
                                    device_id_type=pl.DeviceIdType.MESH)
            return carry

        lax.fori_loop(0, N_DEV - 1, ag_step, 0)

        y = jnp.maximum(out_ref[:, :], 0.0)
        amax = jnp.max(y)
        scale = amax / 127.0
        q = jnp.clip(jnp.round(y / scale), -127.0, 127.0)
        out_ref[:, :] = q * scale

    return pl.pallas_call(
        body,
        out_shape=jax.ShapeDtypeStruct((m, n), jnp.float32),
        in_specs=[
            pl.BlockSpec(memory_space=pltpu.VMEM),
            pl.BlockSpec(memory_space=pltpu.VMEM),
        ],
        out_specs=pl.BlockSpec(memory_space=pltpu.VMEM),
        scratch_shapes=[
            pltpu.VMEM((2, chunk_rows, n), jnp.float32),
            pltpu.SemaphoreType.DMA((2,)),
            pltpu.SemaphoreType.DMA((2,)),
            pltpu.SemaphoreType.REGULAR,
        ],
        compiler_params=pltpu.CompilerParams(collective_id=0),
    )(x, w_mat)


# baseline (device time: 1193135 ns/iter reference)
import jax
import jax.numpy as jnp
from jax import lax
from jax.experimental import pallas as pl
from jax.experimental.pallas import tpu as pltpu

N_DEV = 32


def kernel(x, w_mat):
    m, k_per = x.shape
    _, n = w_mat.shape
    chunk_rows = m // N_DEV

    def body(x_ref, w_ref, out_ref, comm_ref, send_sems, recv_sems, credit_sem):
        my = lax.axis_index("i")
        left = jnp.mod(my - 1, N_DEV)
        right = jnp.mod(my + 1, N_DEV)

        barrier_sem = pltpu.get_barrier_semaphore()
        pl.semaphore_signal(barrier_sem, inc=1, device_id=(left,),
                            device_id_type=pl.DeviceIdType.MESH)
        pl.semaphore_signal(barrier_sem, inc=1, device_id=(right,),
                            device_id_type=pl.DeviceIdType.MESH)
        pl.semaphore_wait(barrier_sem, 2)

        out_ref[:, :] = jnp.dot(x_ref[:, :], w_ref[:, :],
                                preferred_element_type=jnp.float32)

        def out_chunk(idx):
            return out_ref.at[pl.ds(idx * chunk_rows, chunk_rows), :]

        def rs_step(s, carry):
            slot = jnp.mod(s, 2)
            send_idx = jnp.mod(my - s, N_DEV)
            recv_idx = jnp.mod(my - s - 1, N_DEV)

            @pl.when(s > 0)
            def _():
                pl.semaphore_wait(credit_sem, 1)

            rdma = pltpu.make_async_remote_copy(
                src_ref=out_chunk(send_idx),
                dst_ref=comm_ref.at[slot],
                send_sem=send_sems.at[slot],
                recv_sem=recv_sems.at[slot],
                device_id=(right,),
                device_id_type=pl.DeviceIdType.MESH,
            )
            rdma.start()
            rdma.wait()

            out_ref[pl.ds(recv_idx * chunk_rows, chunk_rows), :] += comm_ref[slot]
            pl.semaphore_signal(credit_sem, inc=1, device_id=(left,),
                                device_id_type=pl.DeviceIdType.MESH)
            return carry

        lax.fori_loop(0, N_DEV - 1, rs_step, 0)

        def ag_step(t, carry):
            slot = jnp.mod(t + 1, 2)
            send_idx = jnp.mod(my + 1 - t, N_DEV)

            pl.semaphore_wait(credit_sem, 1)

            rdma = pltpu.make_async_remote_copy(
                src_ref=out_chunk(send_idx),
                dst_ref=out_chunk(send_idx),
                send_sem=send_sems.at[slot],
                recv_sem=recv_sems.at[slot],
                device_id=(right,),
                device_id_type=pl.DeviceIdType.MESH,
            )
            rdma.start()
            rdma.wait()

            @pl.when(t < N_DEV - 2)
            def _():
                pl.semaphore_signal(credit_sem, inc=1, device_id=(left,),
                                    device_id_type=pl.DeviceIdType.MESH)
            return carry

        lax.fori_loop(0, N_DEV - 1, ag_step, 0)

        def amax_step(c, acc):
            ych = jnp.maximum(out_ref[pl.ds(c * chunk_rows, chunk_rows), :], 0.0)
            return jnp.maximum(acc, jnp.max(ych))

        amax = lax.fori_loop(0, N_DEV, amax_step, 0.0)
        scale = amax / 127.0

        def quant_step(c, carry):
            sl = pl.ds(c * chunk_rows, chunk_rows)
            ych = jnp.maximum(out_ref[sl, :], 0.0)
            q = jnp.clip(jnp.round(ych / scale), -127.0, 127.0)
            out_ref[sl, :] = q * scale
            return carry

        lax.fori_loop(0, N_DEV, quant_step, 0)

    return pl.pallas_call(
        body,
        out_shape=jax.ShapeDtypeStruct((m, n), jnp.float32),
        in_specs=[
            pl.BlockSpec(memory_space=pltpu.VMEM),
            pl.BlockSpec(memory_space=pltpu.VMEM),
        ],
        out_specs=pl.BlockSpec(memory_space=pltpu.VMEM),
        scratch_shapes=[
            pltpu.VMEM((2, chunk_rows, n), jnp.float32),
            pltpu.SemaphoreType.DMA((2,)),
            pltpu.SemaphoreType.DMA((2,)),
            pltpu.SemaphoreType.REGULAR,
        ],
        compiler_params=pltpu.CompilerParams(
            collective_id=0,
            vmem_limit_bytes=56 * 1024 * 1024,
        ),
    )(x, w_mat)


# device time: 521869 ns/iter; 2.2863x vs baseline; 2.2863x over previous
import jax
import jax.numpy as jnp
from jax import lax
from jax.experimental import pallas as pl
from jax.experimental.pallas import tpu as pltpu

N_DEV = 32

_PLANE = [(0, 0), (1, 0), (1, 1), (0, 1), (0, 2), (1, 2), (1, 3), (0, 3)]


def _log_to_coords(i):
    x, y = _PLANE[i % 8]
    return (x, y, i // 8)


_YZ_CYCLE = [
    (0, 0), (1, 0), (2, 0), (3, 0),
    (3, 1), (3, 2), (3, 3), (2, 3),
    (2, 2), (2, 1), (1, 1), (1, 2),
    (1, 3), (0, 3), (0, 2), (0, 1),
]
_RING_COORDS = [(0, y, z) for (y, z) in _YZ_CYCLE] + [
    (1, y, z) for (y, z) in reversed(_YZ_CYCLE)
]

_COORDS_TO_LOG = {_log_to_coords(i): i for i in range(N_DEV)}
_RING_LOG = [_COORDS_TO_LOG[c] for c in _RING_COORDS]
_POS_OF_LOG = [0] * N_DEV
for _p, _l in enumerate(_RING_LOG):
    _POS_OF_LOG[_l] = _p
_RIGHT_OF_LOG = [0] * N_DEV
_LEFT_OF_LOG = [0] * N_DEV
for _p, _l in enumerate(_RING_LOG):
    _RIGHT_OF_LOG[_l] = _RING_LOG[(_p + 1) % N_DEV]
    _LEFT_OF_LOG[_l] = _RING_LOG[(_p - 1) % N_DEV]


def kernel(x, w_mat):
    m, k_per = x.shape
    _, n = w_mat.shape
    half = m // 2
    cr = half // N_DEV
    ep_rows = m // N_DEV

    def body(tab_ref, x_ref, w_ref, out_ref,
             comm_a, comm_b, send_a, recv_a, send_b, recv_b,
             credit_a, credit_b):
        my = lax.axis_index("i")

        idx = lax.broadcasted_iota(jnp.int32, (1, N_DEV), 1)

        def lut(row):
            return jnp.sum(jnp.where(idx == my, tab_ref[row:row + 1, :], 0))

        p = lut(0)
        right = lut(1)
        left = lut(2)

        barrier_sem = pltpu.get_barrier_semaphore()
        pl.semaphore_signal(barrier_sem, inc=1, device_id=(left,),
                            device_id_type=pl.DeviceIdType.MESH)
        pl.semaphore_signal(barrier_sem, inc=1, device_id=(right,),
                            device_id_type=pl.DeviceIdType.MESH)
        pl.semaphore_wait(barrier_sem, 2)

        out_ref[:, :] = jnp.dot(x_ref[:, :], w_ref[:, :],
                                preferred_element_type=jnp.float32)

        pl.semaphore_signal(credit_a, inc=1, device_id=(left,),
                            device_id_type=pl.DeviceIdType.MESH)
        pl.semaphore_signal(credit_b, inc=1, device_id=(right,),
                            device_id_type=pl.DeviceIdType.MESH)

        def chunk_a(c):
            return out_ref.at[pl.ds(c * cr, cr), :]

        def chunk_b(c):
            return out_ref.at[pl.ds(half + c * cr, cr), :]

        def rs_step(s, carry):
            slot = jnp.mod(s, 2)
            send_ia = jnp.mod(p - s, N_DEV)
            recv_ia = jnp.mod(p - s - 1, N_DEV)
            send_ib = jnp.mod(p + s, N_DEV)
            recv_ib = jnp.mod(p + s + 1, N_DEV)

            @pl.when(s > 0)
            def _():
                pl.semaphore_wait(credit_a, 1)
                pl.semaphore_wait(credit_b, 1)

            rdma_a = pltpu.make_async_remote_copy(
                src_ref=chunk_a(send_ia),
                dst_ref=comm_a.at[slot],
                send_sem=send_a.at[slot],
                recv_sem=recv_a.at[slot],
                device_id=(right,),
                device_id_type=pl.DeviceIdType.MESH,
            )
            rdma_b = pltpu.make_async_remote_copy(
                src_ref=chunk_b(send_ib),
                dst_ref=comm_b.at[slot],
                send_sem=send_b.at[slot],
                recv_sem=recv_b.at[slot],
                device_id=(left,),
                device_id_type=pl.DeviceIdType.MESH,
            )
            rdma_a.start()
            rdma_b.start()
            rdma_a.wait()
            rdma_b.wait()

            out_ref[pl.ds(recv_ia * cr, cr), :] += comm_a[slot]
            out_ref[pl.ds(half + recv_ib * cr, cr), :] += comm_b[slot]

            pl.semaphore_signal(credit_a, inc=1, device_id=(left,),
                                device_id_type=pl.DeviceIdType.MESH)
            pl.semaphore_signal(credit_b, inc=1, device_id=(right,),
                                device_id_type=pl.DeviceIdType.MESH)
            return carry

        lax.fori_loop(0, N_DEV - 1, rs_step, 0)

        def ag_step(t, carry):
            slot = jnp.mod(t + 1, 2)
            send_ia = jnp.mod(p + 1 - t, N_DEV)
            send_ib = jnp.mod(p - 1 + t, N_DEV)

            pl.semaphore_wait(credit_a, 1)
            pl.semaphore_wait(credit_b, 1)

            rdma_a = pltpu.make_async_remote_copy(
                src_ref=chunk_a(send_ia),
                dst_ref=chunk_a(send_ia),
                send_sem=send_a.at[slot],
                recv_sem=recv_a.at[slot],
                device_id=(right,),
                device_id_type=pl.DeviceIdType.MESH,
            )
            rdma_b = pltpu.make_async_remote_copy(
                src_ref=chunk_b(send_ib),
                dst_ref=chunk_b(send_ib),
                send_sem=send_b.at[slot],
                recv_sem=recv_b.at[slot],
                device_id=(left,),
                device_id_type=pl.DeviceIdType.MESH,
            )
            rdma_a.start()
            rdma_b.start()
            rdma_a.wait()
            rdma_b.wait()

            @pl.when(t < N_DEV - 3)
            def _():
                pl.semaphore_signal(credit_a, inc=1, device_id=(left,),
                                    device_id_type=pl.DeviceIdType.MESH)
                pl.semaphore_signal(credit_b, inc=1, device_id=(right,),
                                    device_id_type=pl.DeviceIdType.MESH)
            return carry

        lax.fori_loop(0, N_DEV - 1, ag_step, 0)

        def amax_step(c, acc):
            ych = jnp.maximum(out_ref[pl.ds(c * ep_rows, ep_rows), :], 0.0)
            return jnp.maximum(acc, jnp.max(ych))

        amax = lax.fori_loop(0, N_DEV, amax_step, 0.0)
        scale = amax / 127.0

        def quant_step(c, carry):
            sl = pl.ds(c * ep_rows, ep_rows)
            ych = jnp.maximum(out_ref[sl, :], 0.0)
            q = jnp.clip(jnp.round(ych / scale), -127.0, 127.0)
            out_ref[sl, :] = q * scale
            return carry

        lax.fori_loop(0, N_DEV, quant_step, 0)

    tables = jnp.array([_POS_OF_LOG, _RIGHT_OF_LOG, _LEFT_OF_LOG],
                       dtype=jnp.int32)

    return pl.pallas_call(
        body,
        out_shape=jax.ShapeDtypeStruct((m, n), jnp.float32),
        in_specs=[
            pl.BlockSpec(memory_space=pltpu.VMEM),
            pl.BlockSpec(memory_space=pltpu.VMEM),
            pl.BlockSpec(memory_space=pltpu.VMEM),
        ],
        out_specs=pl.BlockSpec(memory_space=pltpu.VMEM),
        scratch_shapes=[
            pltpu.VMEM((2, half // N_DEV, n), jnp.float32),
            pltpu.VMEM((2, half // N_DEV, n), jnp.float32),
            pltpu.SemaphoreType.DMA((2,)),
            pltpu.SemaphoreType.DMA((2,)),
            pltpu.SemaphoreType.DMA((2,)),
            pltpu.SemaphoreType.DMA((2,)),
            pltpu.SemaphoreType.REGULAR,
            pltpu.SemaphoreType.REGULAR,
        ],
        compiler_params=pltpu.CompilerParams(
            collective_id=0,
            vmem_limit_bytes=56 * 1024 * 1024,
        ),
    )(tables, x, w_mat)


# device time: 382726 ns/iter; 3.1175x vs baseline; 1.3636x over previous
import jax
import jax.numpy as jnp
from jax import lax
from jax.experimental import pallas as pl
from jax.experimental.pallas import tpu as pltpu

N_DEV = 32

_PLANE = [(0, 0), (1, 0), (1, 1), (0, 1), (0, 2), (1, 2), (1, 3), (0, 3)]


def _log_to_coords(i):
    x, y = _PLANE[i % 8]
    return (x, y, i // 8)


_YZ_CYCLE = [
    (0, 0), (1, 0), (2, 0), (3, 0),
    (3, 1), (3, 2), (3, 3), (2, 3),
    (2, 2), (2, 1), (1, 1), (1, 2),
    (1, 3), (0, 3), (0, 2), (0, 1),
]
_RING_COORDS = [(0, y, z) for (y, z) in _YZ_CYCLE] + [
    (1, y, z) for (y, z) in reversed(_YZ_CYCLE)
]

_COORDS_TO_LOG = {_log_to_coords(i): i for i in range(N_DEV)}
_RING_LOG = [_COORDS_TO_LOG[c] for c in _RING_COORDS]
_POS_OF_LOG = [0] * N_DEV
for _p, _l in enumerate(_RING_LOG):
    _POS_OF_LOG[_l] = _p
_RIGHT_OF_LOG = [0] * N_DEV
_LEFT_OF_LOG = [0] * N_DEV
for _p, _l in enumerate(_RING_LOG):
    _RIGHT_OF_LOG[_l] = _RING_LOG[(_p + 1) % N_DEV]
    _LEFT_OF_LOG[_l] = _RING_LOG[(_p - 1) % N_DEV]


def kernel(x, w_mat):
    m, k_per = x.shape
    _, n = w_mat.shape
    half = m // 2
    cr = half // N_DEV
    ep_rows = m // N_DEV

    def body(tab_ref, x_ref, w_ref, out_ref,
             comm_a, comm_b, send_a, recv_a, send_b, recv_b,
             credit_a, credit_b,
             q_full, amax_src, amax_buf, amax_send, amax_recv):
        my = lax.axis_index("i")

        idx = lax.broadcasted_iota(jnp.int32, (1, N_DEV), 1)

        def lut(row):
            return jnp.sum(jnp.where(idx == my, tab_ref[row:row + 1, :], 0))

        p = lut(0)
        right = lut(1)
        left = lut(2)

        barrier_sem = pltpu.get_barrier_semaphore()
        pl.semaphore_signal(barrier_sem, inc=1, device_id=(left,),
                            device_id_type=pl.DeviceIdType.MESH)
        pl.semaphore_signal(barrier_sem, inc=1, device_id=(right,),
                            device_id_type=pl.DeviceIdType.MESH)
        pl.semaphore_wait(barrier_sem, 2)

        out_ref[:, :] = jnp.dot(x_ref[:, :], w_ref[:, :],
                                preferred_element_type=jnp.float32)

        pl.semaphore_signal(credit_a, inc=1, device_id=(left,),
                            device_id_type=pl.DeviceIdType.MESH)
        pl.semaphore_signal(credit_b, inc=1, device_id=(right,),
                            device_id_type=pl.DeviceIdType.MESH)

        def chunk_a(c):
            return out_ref.at[pl.ds(c * cr, cr), :]

        def chunk_b(c):
            return out_ref.at[pl.ds(half + c * cr, cr), :]

        def rs_step(s, carry):
            slot = jnp.mod(s, 2)
            send_ia = jnp.mod(p - s, N_DEV)
            recv_ia = jnp.mod(p - s - 1, N_DEV)
            send_ib = jnp.mod(p + s, N_DEV)
            recv_ib = jnp.mod(p + s + 1, N_DEV)

            @pl.when(s > 0)
            def _():
                pl.semaphore_wait(credit_a, 1)
                pl.semaphore_wait(credit_b, 1)

            rdma_a = pltpu.make_async_remote_copy(
                src_ref=chunk_a(send_ia),
                dst_ref=comm_a.at[slot],
                send_sem=send_a.at[slot],
                recv_sem=recv_a.at[slot],
                device_id=(right,),
                device_id_type=pl.DeviceIdType.MESH,
            )
            rdma_b = pltpu.make_async_remote_copy(
                src_ref=chunk_b(send_ib),
                dst_ref=comm_b.at[slot],
                send_sem=send_b.at[slot],
                recv_sem=recv_b.at[slot],
                device_id=(left,),
                device_id_type=pl.DeviceIdType.MESH,
            )
            rdma_a.start()
            rdma_b.start()
            rdma_a.wait()
            rdma_b.wait()

            out_ref[pl.ds(recv_ia * cr, cr), :] += comm_a[slot]
            out_ref[pl.ds(half + recv_ib * cr, cr), :] += comm_b[slot]

            pl.semaphore_signal(credit_a, inc=1, device_id=(left,),
                                device_id_type=pl.DeviceIdType.MESH)
            pl.semaphore_signal(credit_b, inc=1, device_id=(right,),
                                device_id_type=pl.DeviceIdType.MESH)
            return carry

        lax.fori_loop(0, N_DEV - 1, rs_step, 0)

        ra = jnp.mod(p + 1, N_DEV)
        rb = jnp.mod(p - 1, N_DEV)
        ya = jnp.maximum(out_ref[pl.ds(ra * cr, cr), :], 0.0)
        yb = jnp.maximum(out_ref[pl.ds(half + rb * cr, cr), :], 0.0)
        amax_local = jnp.maximum(jnp.max(ya), jnp.max(yb))
        amax_src[0, :] = jnp.full((128,), amax_local, jnp.float32)
        amax_buf[pl.ds(my, 1), :] = amax_src[0:1, :]

        def bcast_step(o, carry):
            tgt = jnp.mod(my + o, N_DEV)
            rdma = pltpu.make_async_remote_copy(
                src_ref=amax_src.at[0],
                dst_ref=amax_buf.at[my],
                send_sem=amax_send.at[o],
                recv_sem=amax_recv.at[my],
                device_id=(tgt,),
                device_id_type=pl.DeviceIdType.MESH,
            )
            rdma.start()
            return carry

        lax.fori_loop(1, N_DEV, bcast_step, 0)

        def bcast_wait(o, carry):
            src_id = jnp.mod(my + o, N_DEV)
            rdma = pltpu.make_async_remote_copy(
                src_ref=amax_src.at[0],
                dst_ref=amax_buf.at[src_id],
                send_sem=amax_send.at[o],
                recv_sem=amax_recv.at[src_id],
                device_id=(src_id,),
                device_id_type=pl.DeviceIdType.MESH,
            )
            rdma.wait_recv()
            return carry

        lax.fori_loop(1, N_DEV, bcast_wait, 0)

        amax = jnp.max(amax_buf[:, :])
        scale = amax / 127.0

        def q8(y):
            return jnp.clip(jnp.round(y / scale), -127.0, 127.0).astype(jnp.int8)

        q_full[pl.ds(ra * cr, cr), :] = q8(ya)
        q_full[pl.ds(half + rb * cr, cr), :] = q8(yb)

        def q_chunk_a(c):
            return q_full.at[pl.ds(c * cr, cr), :]

        def q_chunk_b(c):
            return q_full.at[pl.ds(half + c * cr, cr), :]

        def bcast_drain(o, carry):
            tgt = jnp.mod(my + o, N_DEV)
            rdma = pltpu.make_async_remote_copy(
                src_ref=amax_src.at[0],
                dst_ref=amax_buf.at[my],
                send_sem=amax_send.at[o],
                recv_sem=amax_recv.at[my],
                device_id=(tgt,),
                device_id_type=pl.DeviceIdType.MESH,
            )
            rdma.wait_send()
            return carry

        lax.fori_loop(1, N_DEV, bcast_drain, 0)

        def ag_step(t, carry):
            slot = jnp.mod(t + 1, 2)
            send_ia = jnp.mod(p + 1 - t, N_DEV)
            send_ib = jnp.mod(p - 1 + t, N_DEV)

            pl.semaphore_wait(credit_a, 1)
            pl.semaphore_wait(credit_b, 1)

            rdma_a = pltpu.make_async_remote_copy(
                src_ref=q_chunk_a(send_ia),
                dst_ref=q_chunk_a(send_ia),
                send_sem=send_a.at[slot],
                recv_sem=recv_a.at[slot],
                device_id=(right,),
                device_id_type=pl.DeviceIdType.MESH,
            )
            rdma_b = pltpu.make_async_remote_copy(
                src_ref=q_chunk_b(send_ib),
                dst_ref=q_chunk_b(send_ib),
                send_sem=send_b.at[slot],
                recv_sem=recv_b.at[slot],
                device_id=(left,),
                device_id_type=pl.DeviceIdType.MESH,
            )
            rdma_a.start()
            rdma_b.start()
            rdma_a.wait()
            rdma_b.wait()

            @pl.when(t < N_DEV - 3)
            def _():
                pl.semaphore_signal(credit_a, inc=1, device_id=(left,),
                                    device_id_type=pl.DeviceIdType.MESH)
                pl.semaphore_signal(credit_b, inc=1, device_id=(right,),
                                    device_id_type=pl.DeviceIdType.MESH)
            return carry

        lax.fori_loop(0, N_DEV - 1, ag_step, 0)

        def deq_step(c, carry):
            sl = pl.ds(c * ep_rows, ep_rows)
            out_ref[sl, :] = q_full[sl, :].astype(jnp.float32) * scale
            return carry

        lax.fori_loop(0, N_DEV, deq_step, 0)

    tables = jnp.array([_POS_OF_LOG, _RIGHT_OF_LOG, _LEFT_OF_LOG],
                       dtype=jnp.int32)

    return pl.pallas_call(
        body,
        out_shape=jax.ShapeDtypeStruct((m, n), jnp.float32),
        in_specs=[
            pl.BlockSpec(memory_space=pltpu.VMEM),
            pl.BlockSpec(memory_space=pltpu.VMEM),
            pl.BlockSpec(memory_space=pltpu.VMEM),
        ],
        out_specs=pl.BlockSpec(memory_space=pltpu.VMEM),
        scratch_shapes=[
            pltpu.VMEM((2, half // N_DEV, n), jnp.float32),
            pltpu.VMEM((2, half // N_DEV, n), jnp.float32),
            pltpu.SemaphoreType.DMA((2,)),
            pltpu.SemaphoreType.DMA((2,)),
            pltpu.SemaphoreType.DMA((2,)),
            pltpu.SemaphoreType.DMA((2,)),
            pltpu.SemaphoreType.REGULAR,
            pltpu.SemaphoreType.REGULAR,
            pltpu.VMEM((m, n), jnp.int8),
            pltpu.VMEM((1, 128), jnp.float32),
            pltpu.VMEM((N_DEV, 128), jnp.float32),
            pltpu.SemaphoreType.DMA((N_DEV,)),
            pltpu.SemaphoreType.DMA((N_DEV,)),
        ],
        compiler_params=pltpu.CompilerParams(
            collective_id=0,
            vmem_limit_bytes=56 * 1024 * 1024,
        ),
    )(tables, x, w_mat)


# device time: 374776 ns/iter; 3.1836x vs baseline; 1.0212x over previous
import jax
import jax.numpy as jnp
from jax import lax
from jax.experimental import pallas as pl
from jax.experimental.pallas import tpu as pltpu

N_DEV = 32

_PLANE = [(0, 0), (1, 0), (1, 1), (0, 1), (0, 2), (1, 2), (1, 3), (0, 3)]


def _log_to_coords(i):
    x, y = _PLANE[i % 8]
    return (x, y, i // 8)


_YZ_CYCLE = [
    (0, 0), (1, 0), (2, 0), (3, 0),
    (3, 1), (3, 2), (3, 3), (2, 3),
    (2, 2), (2, 1), (1, 1), (1, 2),
    (1, 3), (0, 3), (0, 2), (0, 1),
]
_RING_COORDS = [(0, y, z) for (y, z) in _YZ_CYCLE] + [
    (1, y, z) for (y, z) in reversed(_YZ_CYCLE)
]

_COORDS_TO_LOG = {_log_to_coords(i): i for i in range(N_DEV)}
_RING_LOG = [_COORDS_TO_LOG[c] for c in _RING_COORDS]
_POS_OF_LOG = [0] * N_DEV
for _p, _l in enumerate(_RING_LOG):
    _POS_OF_LOG[_l] = _p
_RIGHT_OF_LOG = [0] * N_DEV
_LEFT_OF_LOG = [0] * N_DEV
for _p, _l in enumerate(_RING_LOG):
    _RIGHT_OF_LOG[_l] = _RING_LOG[(_p + 1) % N_DEV]
    _LEFT_OF_LOG[_l] = _RING_LOG[(_p - 1) % N_DEV]


def kernel(x, w_mat):
    m, k_per = x.shape
    _, n = w_mat.shape
    half = m // 2
    cr = half // N_DEV
    ep_rows = m // N_DEV

    def body(tab_ref, x_ref, w_ref, out_ref,
             comm_a, comm_b, send_a, recv_a, send_b, recv_b,
             credit_a, credit_b,
             q_full, amax_src, amax_buf, amax_send, amax_recv):
        my = lax.axis_index("i")

        idx = lax.broadcasted_iota(jnp.int32, (1, N_DEV), 1)

        def lut(row):
            return jnp.sum(jnp.where(idx == my, tab_ref[row:row + 1, :], 0))

        p = lut(0)
        right = lut(1)
        left = lut(2)

        barrier_sem = pltpu.get_barrier_semaphore()
        pl.semaphore_signal(barrier_sem, inc=1, device_id=(left,),
                            device_id_type=pl.DeviceIdType.MESH)
        pl.semaphore_signal(barrier_sem, inc=1, device_id=(right,),
                            device_id_type=pl.DeviceIdType.MESH)
        pl.semaphore_wait(barrier_sem, 2)

        def gemm_rows(r0):
            out_ref[pl.ds(r0, cr), :] = jnp.dot(
                x_ref[pl.ds(r0, cr), :], w_ref[:, :],
                preferred_element_type=jnp.float32)

        pl.semaphore_signal(credit_a, inc=1, device_id=(left,),
                            device_id_type=pl.DeviceIdType.MESH)
        pl.semaphore_signal(credit_b, inc=1, device_id=(right,),
                            device_id_type=pl.DeviceIdType.MESH)

        def chunk_a(c):
            return out_ref.at[pl.ds(c * cr, cr), :]

        def chunk_b(c):
            return out_ref.at[pl.ds(half + c * cr, cr), :]

        gemm_rows(p * cr)
        gemm_rows(half + p * cr)

        def rs_step(s, carry):
            slot = jnp.mod(s, 2)
            send_ia = jnp.mod(p - s, N_DEV)
            recv_ia = jnp.mod(p - s - 1, N_DEV)
            send_ib = jnp.mod(p + s, N_DEV)
            recv_ib = jnp.mod(p + s + 1, N_DEV)

            @pl.when(s > 0)
            def _():
                pl.semaphore_wait(credit_a, 1)
                pl.semaphore_wait(credit_b, 1)

            rdma_a = pltpu.make_async_remote_copy(
                src_ref=chunk_a(send_ia),
                dst_ref=comm_a.at[slot],
                send_sem=send_a.at[slot],
                recv_sem=recv_a.at[slot],
                device_id=(right,),
                device_id_type=pl.DeviceIdType.MESH,
            )
            rdma_b = pltpu.make_async_remote_copy(
                src_ref=chunk_b(send_ib),
                dst_ref=comm_b.at[slot],
                send_sem=send_b.at[slot],
                recv_sem=recv_b.at[slot],
                device_id=(left,),
                device_id_type=pl.DeviceIdType.MESH,
            )
            rdma_a.start()
            rdma_b.start()

            gemm_rows(recv_ia * cr)
            gemm_rows(half + recv_ib * cr)

            rdma_a.wait()
            rdma_b.wait()

            out_ref[pl.ds(recv_ia * cr, cr), :] += comm_a[slot]
            out_ref[pl.ds(half + recv_ib * cr, cr), :] += comm_b[slot]

            pl.semaphore_signal(credit_a, inc=1, device_id=(left,),
                                device_id_type=pl.DeviceIdType.MESH)
            pl.semaphore_signal(credit_b, inc=1, device_id=(right,),
                                device_id_type=pl.DeviceIdType.MESH)
            return carry

        lax.fori_loop(0, N_DEV - 1, rs_step, 0)

        ra = jnp.mod(p + 1, N_DEV)
        rb = jnp.mod(p - 1, N_DEV)
        ya = jnp.maximum(out_ref[pl.ds(ra * cr, cr), :], 0.0)
        yb = jnp.maximum(out_ref[pl.ds(half + rb * cr, cr), :], 0.0)
        amax_local = jnp.maximum(jnp.max(ya), jnp.max(yb))
        amax_src[0, :] = jnp.full((128,), amax_local, jnp.float32)
        amax_buf[pl.ds(my, 1), :] = amax_src[0:1, :]

        def bcast_step(o, carry):
            tgt = jnp.mod(my + o, N_DEV)
            rdma = pltpu.make_async_remote_copy(
                src_ref=amax_src.at[0],
                dst_ref=amax_buf.at[my],
                send_sem=amax_send.at[o],
                recv_sem=amax_recv.at[my],
                device_id=(tgt,),
                device_id_type=pl.DeviceIdType.MESH,
            )
            rdma.start()
            return carry

        lax.fori_loop(1, N_DEV, bcast_step, 0)

        def bcast_wait(o, carry):
            src_id = jnp.mod(my + o, N_DEV)
            rdma = pltpu.make_async_remote_copy(
                src_ref=amax_src.at[0],
                dst_ref=amax_buf.at[src_id],
                send_sem=amax_send.at[o],
                recv_sem=amax_recv.at[src_id],
                device_id=(src_id,),
                device_id_type=pl.DeviceIdType.MESH,
            )
            rdma.wait_recv()
            return carry

        lax.fori_loop(1, N_DEV, bcast_wait, 0)

        amax = jnp.max(amax_buf[:, :])
        scale = amax / 127.0

        def q8(y):
            return jnp.clip(jnp.round(y / scale), -127.0, 127.0).astype(jnp.int8)

        q_full[pl.ds(ra * cr, cr), :] = q8(ya)
        q_full[pl.ds(half + rb * cr, cr), :] = q8(yb)

        def q_chunk_a(c):
            return q_full.at[pl.ds(c * cr, cr), :]

        def q_chunk_b(c):
            return q_full.at[pl.ds(half + c * cr, cr), :]

        def deq_a(c):
            sl = pl.ds(c * cr, cr)
            out_ref[sl, :] = q_full[sl, :].astype(jnp.float32) * scale

        def deq_b(c):
            sl = pl.ds(half + c * cr, cr)
            out_ref[sl, :] = q_full[sl, :].astype(jnp.float32) * scale

        def bcast_drain(o, carry):
            tgt = jnp.mod(my + o, N_DEV)
            rdma = pltpu.make_async_remote_copy(
                src_ref=amax_src.at[0],
                dst_ref=amax_buf.at[my],
                send_sem=amax_send.at[o],
                recv_sem=amax_recv.at[my],
                device_id=(tgt,),
                device_id_type=pl.DeviceIdType.MESH,
            )
            rdma.wait_send()
            return carry

        lax.fori_loop(1, N_DEV, bcast_drain, 0)

        def ag_step(t, carry):
            slot = jnp.mod(t + 1, 2)
            send_ia = jnp.mod(p + 1 - t, N_DEV)
            send_ib = jnp.mod(p - 1 + t, N_DEV)

            pl.semaphore_wait(credit_a, 1)
            pl.semaphore_wait(credit_b, 1)

            rdma_a = pltpu.make_async_remote_copy(
                src_ref=q_chunk_a(send_ia),
                dst_ref=q_chunk_a(send_ia),
                send_sem=send_a.at[slot],
                recv_sem=recv_a.at[slot],
                device_id=(right,),
                device_id_type=pl.DeviceIdType.MESH,
            )
            rdma_b = pltpu.make_async_remote_copy(
                src_ref=q_chunk_b(send_ib),
                dst_ref=q_chunk_b(send_ib),
                send_sem=send_b.at[slot],
                recv_sem=recv_b.at[slot],
                device_id=(left,),
                device_id_type=pl.DeviceIdType.MESH,
            )
            rdma_a.start()
            rdma_b.start()

            @pl.when(t == 0)
            def _():
                deq_a(ra)
                deq_b(rb)

            @pl.when(t > 0)
            def _():
                deq_a(jnp.mod(p - t + 1, N_DEV))
                deq_b(jnp.mod(p + t - 1, N_DEV))

            rdma_a.wait()
            rdma_b.wait()

            @pl.when(t < N_DEV - 3)
            def _():
                pl.semaphore_signal(credit_a, inc=1, device_id=(left,),
                                    device_id_type=pl.DeviceIdType.MESH)
                pl.semaphore_signal(credit_b, inc=1, device_id=(right,),
                                    device_id_type=pl.DeviceIdType.MESH)
            return carry

        lax.fori_loop(0, N_DEV - 1, ag_step, 0)

        deq_a(jnp.mod(p + 2, N_DEV))
        deq_b(jnp.mod(p - 2, N_DEV))

    tables = jnp.array([_POS_OF_LOG, _RIGHT_OF_LOG, _LEFT_OF_LOG],
                       dtype=jnp.int32)

    return pl.pallas_call(
        body,
        out_shape=jax.ShapeDtypeStruct((m, n), jnp.float32),
        in_specs=[
            pl.BlockSpec(memory_space=pltpu.VMEM),
            pl.BlockSpec(memory_space=pltpu.VMEM),
            pl.BlockSpec(memory_space=pltpu.VMEM),
        ],
        out_specs=pl.BlockSpec(memory_space=pltpu.VMEM),
        scratch_shapes=[
            pltpu.VMEM((2, half // N_DEV, n), jnp.float32),
            pltpu.VMEM((2, half // N_DEV, n), jnp.float32),
            pltpu.SemaphoreType.DMA((2,)),
            pltpu.SemaphoreType.DMA((2,)),
            pltpu.SemaphoreType.DMA((2,)),
            pltpu.SemaphoreType.DMA((2,)),
            pltpu.SemaphoreType.REGULAR,
            pltpu.SemaphoreType.REGULAR,
            pltpu.VMEM((m, n), jnp.int8),
            pltpu.VMEM((1, 128), jnp.float32),
            pltpu.VMEM((N_DEV, 128), jnp.float32),
            pltpu.SemaphoreType.DMA((N_DEV,)),
            pltpu.SemaphoreType.DMA((N_DEV,)),
        ],
        compiler_params=pltpu.CompilerParams(
            collective_id=0,
            vmem_limit_bytes=56 * 1024 * 1024,
        ),
    )(tables, x, w_mat)


# device time: 290429 ns/iter; 4.1082x vs baseline; 1.2904x over previous
import jax
import jax.numpy as jnp
from jax import lax
from jax.experimental import pallas as pl
from jax.experimental.pallas import tpu as pltpu

N_DEV = 32

_PLANE = [(0, 0), (1, 0), (1, 1), (0, 1), (0, 2), (1, 2), (1, 3), (0, 3)]


def _log_to_coords(i):
    x, y = _PLANE[i % 8]
    return (x, y, i // 8)


_YZ_CYCLE = [
    (0, 0), (1, 0), (2, 0), (3, 0),
    (3, 1), (3, 2), (3, 3), (2, 3),
    (2, 2), (2, 1), (1, 1), (1, 2),
    (1, 3), (0, 3), (0, 2), (0, 1),
]
_RING_COORDS = [(0, y, z) for (y, z) in _YZ_CYCLE] + [
    (1, y, z) for (y, z) in reversed(_YZ_CYCLE)
]

_COORDS_TO_LOG = {_log_to_coords(i): i for i in range(N_DEV)}
_RING_LOG = [_COORDS_TO_LOG[c] for c in _RING_COORDS]
_POS_OF_LOG = [0] * N_DEV
for _p, _l in enumerate(_RING_LOG):
    _POS_OF_LOG[_l] = _p
_RIGHT_OF_LOG = [0] * N_DEV
_LEFT_OF_LOG = [0] * N_DEV
for _p, _l in enumerate(_RING_LOG):
    _RIGHT_OF_LOG[_l] = _RING_LOG[(_p + 1) % N_DEV]
    _LEFT_OF_LOG[_l] = _RING_LOG[(_p - 1) % N_DEV]


def kernel(x, w_mat):
    m, k_per = x.shape
    _, n = w_mat.shape
    half = m // 2
    cr = half // N_DEV
    ep_rows = m // N_DEV

    def body(tab_ref, x_ref, w_ref, out_ref,
             comm_a, comm_b, sbuf_a, sbuf_b,
             send_a, recv_a, send_b, recv_b,
             credit_a, credit_b,
             q_full, amax_src, amax_buf, amax_send, amax_recv):
        my = lax.axis_index("i")

        idx = lax.broadcasted_iota(jnp.int32, (1, N_DEV), 1)

        def lut(row):
            return jnp.sum(jnp.where(idx == my, tab_ref[row:row + 1, :], 0))

        p = lut(0)
        right = lut(1)
        left = lut(2)

        barrier_sem = pltpu.get_barrier_semaphore()
        pl.semaphore_signal(barrier_sem, inc=1, device_id=(left,),
                            device_id_type=pl.DeviceIdType.MESH)
        pl.semaphore_signal(barrier_sem, inc=1, device_id=(right,),
                            device_id_type=pl.DeviceIdType.MESH)
        pl.semaphore_wait(barrier_sem, 2)

        def gemm_rows(r0):
            out_ref[pl.ds(r0, cr), :] = jnp.dot(
                x_ref[pl.ds(r0, cr), :], w_ref[:, :],
                preferred_element_type=jnp.float32)

        pl.semaphore_signal(credit_a, inc=1, device_id=(left,),
                            device_id_type=pl.DeviceIdType.MESH)
        pl.semaphore_signal(credit_b, inc=1, device_id=(right,),
                            device_id_type=pl.DeviceIdType.MESH)

        def chunk_a(c):
            return out_ref.at[pl.ds(c * cr, cr), :]

        def chunk_b(c):
            return out_ref.at[pl.ds(half + c * cr, cr), :]

        gemm_rows(p * cr)
        gemm_rows(half + p * cr)
        sbuf_a[0] = out_ref[pl.ds(p * cr, cr), :].astype(jnp.bfloat16)
        sbuf_b[0] = out_ref[pl.ds(half + p * cr, cr), :].astype(jnp.bfloat16)

        def rs_step(s, carry):
            slot = jnp.mod(s, 2)
            nslot = jnp.mod(s + 1, 2)
            recv_ia = jnp.mod(p - s - 1, N_DEV)
            recv_ib = jnp.mod(p + s + 1, N_DEV)

            @pl.when(s > 0)
            def _():
                pl.semaphore_wait(credit_a, 1)
                pl.semaphore_wait(credit_b, 1)

            rdma_a = pltpu.make_async_remote_copy(
                src_ref=sbuf_a.at[slot],
                dst_ref=comm_a.at[slot],
                send_sem=send_a.at[slot],
                recv_sem=recv_a.at[slot],
                device_id=(right,),
                device_id_type=pl.DeviceIdType.MESH,
            )
            rdma_b = pltpu.make_async_remote_copy(
                src_ref=sbuf_b.at[slot],
                dst_ref=comm_b.at[slot],
                send_sem=send_b.at[slot],
                recv_sem=recv_b.at[slot],
                device_id=(left,),
                device_id_type=pl.DeviceIdType.MESH,
            )
            rdma_a.start()
            rdma_b.start()

            gemm_rows(recv_ia * cr)
            gemm_rows(half + recv_ib * cr)

            rdma_a.wait()
            rdma_b.wait()

            sum_a = out_ref[pl.ds(recv_ia * cr, cr), :] + \
                comm_a[slot].astype(jnp.float32)
            out_ref[pl.ds(recv_ia * cr, cr), :] = sum_a
            sum_b = out_ref[pl.ds(half + recv_ib * cr, cr), :] + \
                comm_b[slot].astype(jnp.float32)
            out_ref[pl.ds(half + recv_ib * cr, cr), :] = sum_b

            @pl.when(s < N_DEV - 2)
            def _():
                sbuf_a[pl.ds(nslot, 1)] = sum_a.astype(jnp.bfloat16)[None]
                sbuf_b[pl.ds(nslot, 1)] = sum_b.astype(jnp.bfloat16)[None]

            pl.semaphore_signal(credit_a, inc=1, device_id=(left,),
                                device_id_type=pl.DeviceIdType.MESH)
            pl.semaphore_signal(credit_b, inc=1, device_id=(right,),
                                device_id_type=pl.DeviceIdType.MESH)
            return carry

        lax.fori_loop(0, N_DEV - 1, rs_step, 0)

        ra = jnp.mod(p + 1, N_DEV)
        rb = jnp.mod(p - 1, N_DEV)
        ya = jnp.maximum(out_ref[pl.ds(ra * cr, cr), :], 0.0)
        yb = jnp.maximum(out_ref[pl.ds(half + rb * cr, cr), :], 0.0)
        amax_local = jnp.maximum(jnp.max(ya), jnp.max(yb))
        amax_src[0, :] = jnp.full((128,), amax_local, jnp.float32)
        amax_buf[pl.ds(my, 1), :] = amax_src[0:1, :]

        def bcast_step(o, carry):
            tgt = jnp.mod(my + o, N_DEV)
            rdma = pltpu.make_async_remote_copy(
                src_ref=amax_src.at[0],
                dst_ref=amax_buf.at[my],
                send_sem=amax_send.at[o],
                recv_sem=amax_recv.at[my],
                device_id=(tgt,),
                device_id_type=pl.DeviceIdType.MESH,
            )
            rdma.start()
            return carry

        lax.fori_loop(1, N_DEV, bcast_step, 0)

        def bcast_wait(o, carry):
            src_id = jnp.mod(my + o, N_DEV)
            rdma = pltpu.make_async_remote_copy(
                src_ref=amax_src.at[0],
                dst_ref=amax_buf.at[src_id],
                send_sem=amax_send.at[o],
                recv_sem=amax_recv.at[src_id],
                device_id=(src_id,),
                device_id_type=pl.DeviceIdType.MESH,
            )
            rdma.wait_recv()
            return carry

        lax.fori_loop(1, N_DEV, bcast_wait, 0)

        amax = jnp.max(amax_buf[:, :])
        scale = amax / 127.0

        def q8(y):
            return jnp.clip(jnp.round(y / scale), -127.0, 127.0).astype(jnp.int8)

        q_full[pl.ds(ra * cr, cr), :] = q8(ya)
        q_full[pl.ds(half + rb * cr, cr), :] = q8(yb)

        def q_chunk_a(c):
            return q_full.at[pl.ds(c * cr, cr), :]

        def q_chunk_b(c):
            return q_full.at[pl.ds(half + c * cr, cr), :]

        def deq_a(c):
            sl = pl.ds(c * cr, cr)
            out_ref[sl, :] = q_full[sl, :].astype(jnp.float32) * scale

        def deq_b(c):
            sl = pl.ds(half + c * cr, cr)
            out_ref[sl, :] = q_full[sl, :].astype(jnp.float32) * scale

        def bcast_drain(o, carry):
            tgt = jnp.mod(my + o, N_DEV)
            rdma = pltpu.make_async_remote_copy(
                src_ref=amax_src.at[0],
                dst_ref=amax_buf.at[my],
                send_sem=amax_send.at[o],
                recv_sem=amax_recv.at[my],
                device_id=(tgt,),
                device_id_type=pl.DeviceIdType.MESH,
            )
            rdma.wait_send()
            return carry

        lax.fori_loop(1, N_DEV, bcast_drain, 0)

        def ag_step(t, carry):
            slot = jnp.mod(t + 1, 2)
            send_ia = jnp.mod(p + 1 - t, N_DEV)
            send_ib = jnp.mod(p - 1 + t, N_DEV)

            pl.semaphore_wait(credit_a, 1)
            pl.semaphore_wait(credit_b, 1)

            rdma_a = pltpu.make_async_remote_copy(
                src_ref=q_chunk_a(send_ia),
                dst_ref=q_chunk_a(send_ia),
                send_sem=send_a.at[slot],
                recv_sem=recv_a.at[slot],
                device_id=(right,),
                device_id_type=pl.DeviceIdType.MESH,
            )
            rdma_b = pltpu.make_async_remote_copy(
                src_ref=q_chunk_b(send_ib),
                dst_ref=q_chunk_b(send_ib),
                send_sem=send_b.at[slot],
                recv_sem=recv_b.at[slot],
                device_id=(left,),
                device_id_type=pl.DeviceIdType.MESH,
            )
            rdma_a.start()
            rdma_b.start()

            @pl.when(t == 0)
            def _():
                deq_a(ra)
                deq_b(rb)

            @pl.when(t > 0)
            def _():
                deq_a(jnp.mod(p - t + 1, N_DEV))
                deq_b(jnp.mod(p + t - 1, N_DEV))

            rdma_a.wait()
            rdma_b.wait()

            @pl.when(t < N_DEV - 3)
            def _():
                pl.semaphore_signal(credit_a, inc=1, device_id=(left,),
                                    device_id_type=pl.DeviceIdType.MESH)
                pl.semaphore_signal(credit_b, inc=1, device_id=(right,),
                                    device_id_type=pl.DeviceIdType.MESH)
            return carry

        lax.fori_loop(0, N_DEV - 1, ag_step, 0)

        deq_a(jnp.mod(p + 2, N_DEV))
        deq_b(jnp.mod(p - 2, N_DEV))

    tables = jnp.array([_POS_OF_LOG, _RIGHT_OF_LOG, _LEFT_OF_LOG],
                       dtype=jnp.int32)

    return pl.pallas_call(
        body,
        out_shape=jax.ShapeDtypeStruct((m, n), jnp.float32),
        in_specs=[
            pl.BlockSpec(memory_space=pltpu.VMEM),
            pl.BlockSpec(memory_space=pltpu.VMEM),
            pl.BlockSpec(memory_space=pltpu.VMEM),
        ],
        out_specs=pl.BlockSpec(memory_space=pltpu.VMEM),
        scratch_shapes=[
            pltpu.VMEM((2, half // N_DEV, n), jnp.bfloat16),
            pltpu.VMEM((2, half // N_DEV, n), jnp.bfloat16),
            pltpu.VMEM((2, half // N_DEV, n), jnp.bfloat16),
            pltpu.VMEM((2, half // N_DEV, n), jnp.bfloat16),
            pltpu.SemaphoreType.DMA((2,)),
            pltpu.SemaphoreType.DMA((2,)),
            pltpu.SemaphoreType.DMA((2,)),
            pltpu.SemaphoreType.DMA((2,)),
            pltpu.SemaphoreType.REGULAR,
            pltpu.SemaphoreType.REGULAR,
            pltpu.VMEM((m, n), jnp.int8),
            pltpu.VMEM((1, 128), jnp.float32),
            pltpu.VMEM((N_DEV, 128), jnp.float32),
            pltpu.SemaphoreType.DMA((N_DEV,)),
            pltpu.SemaphoreType.DMA((N_DEV,)),
        ],
        compiler_params=pltpu.CompilerParams(
            collective_id=0,
            vmem_limit_bytes=56 * 1024 * 1024,
        ),
    )(tables, x, w_mat)


# device time: 236013 ns/iter; 5.0554x vs baseline; 1.2306x over previous
import jax
import jax.numpy as jnp
from jax import lax
from jax.experimental import pallas as pl
from jax.experimental.pallas import tpu as pltpu

N_DEV = 32

_PLANE = [(0, 0), (1, 0), (1, 1), (0, 1), (0, 2), (1, 2), (1, 3), (0, 3)]


def _log_to_coords(i):
    x, y = _PLANE[i % 8]
    return (x, y, i // 8)


_YZ_CYCLE = [
    (0, 0), (1, 0), (2, 0), (3, 0),
    (3, 1), (3, 2), (3, 3), (2, 3),
    (2, 2), (2, 1), (1, 1), (1, 2),
    (1, 3), (0, 3), (0, 2), (0, 1),
]
_RING_COORDS = [(0, y, z) for (y, z) in _YZ_CYCLE] + [
    (1, y, z) for (y, z) in reversed(_YZ_CYCLE)
]

_COORDS_TO_LOG = {_log_to_coords(i): i for i in range(N_DEV)}
_RING_LOG = [_COORDS_TO_LOG[c] for c in _RING_COORDS]
_POS_OF_LOG = [0] * N_DEV
for _p, _l in enumerate(_RING_LOG):
    _POS_OF_LOG[_l] = _p
_RIGHT_OF_LOG = [0] * N_DEV
_LEFT_OF_LOG = [0] * N_DEV
for _p, _l in enumerate(_RING_LOG):
    _RIGHT_OF_LOG[_l] = _RING_LOG[(_p + 1) % N_DEV]
    _LEFT_OF_LOG[_l] = _RING_LOG[(_p - 1) % N_DEV]

H = N_DEV // 2


def kernel(x, w_mat):
    m, k_per = x.shape
    _, n = w_mat.shape
    half = m // 2
    cr = half // N_DEV

    def body(tab_ref, x_ref, w_ref, out_ref,
             comm_acw, comm_accw, comm_bcw, comm_bccw,
             sbuf_acw, sbuf_accw, sbuf_bcw, sbuf_bccw,
             s1s, s1r, s2s, s2r, s3s, s3r, s4s, s4r,
             credit_r, credit_l, credit_r2, credit_l2,
             q_full, amax_src, amax_buf, amax_send, amax_recv):
        my = lax.axis_index("i")

        idx = lax.broadcasted_iota(jnp.int32, (1, N_DEV), 1)

        def lut(row):
            return jnp.sum(jnp.where(idx == my, tab_ref[row:row + 1, :], 0))

        p = lut(0)
        right = lut(1)
        left = lut(2)

        def md(v):
            return jnp.mod(v, N_DEV)

        barrier_sem = pltpu.get_barrier_semaphore()
        pl.semaphore_signal(barrier_sem, inc=1, device_id=(left,),
                            device_id_type=pl.DeviceIdType.MESH)
        pl.semaphore_signal(barrier_sem, inc=1, device_id=(right,),
                            device_id_type=pl.DeviceIdType.MESH)
        pl.semaphore_wait(barrier_sem, 2)

        pl.semaphore_signal(credit_r, inc=1, device_id=(left,),
                            device_id_type=pl.DeviceIdType.MESH)
        pl.semaphore_signal(credit_l, inc=1, device_id=(right,),
                            device_id_type=pl.DeviceIdType.MESH)
        pl.semaphore_signal(credit_r2, inc=1, device_id=(left,),
                            device_id_type=pl.DeviceIdType.MESH)
        pl.semaphore_signal(credit_l2, inc=1, device_id=(right,),
                            device_id_type=pl.DeviceIdType.MESH)

        def a_rows(c):
            return pl.ds(c * cr, cr)

        def b_rows(c):
            return pl.ds(half + c * cr, cr)

        def gemm_a(c):
            out_ref[a_rows(c), :] = jnp.dot(
                x_ref[a_rows(c), :], w_ref[:, :],
                preferred_element_type=jnp.float32)

        def gemm_b(c):
            out_ref[b_rows(c), :] = jnp.dot(
                x_ref[b_rows(c), :], w_ref[:, :],
                preferred_element_type=jnp.float32)

        gemm_a(md(p + 17))
        gemm_a(md(p - 14))
        gemm_b(md(p - 17))
        gemm_b(md(p + 14))
        sbuf_acw[0] = out_ref[a_rows(md(p + 17)), :].astype(jnp.bfloat16)
        sbuf_accw[0] = out_ref[a_rows(md(p - 14)), :].astype(jnp.bfloat16)
        sbuf_bcw[0] = out_ref[b_rows(md(p - 17)), :].astype(jnp.bfloat16)
        sbuf_bccw[0] = out_ref[b_rows(md(p + 14)), :].astype(jnp.bfloat16)

        def rs_step(s, carry):
            slot = jnp.mod(s, 2)
            nslot = jnp.mod(s + 1, 2)
            ca_cw = md(p + 16 - s)
            ca_ccw = md(p - 13 + s)
            cb_cw = md(p - 16 + s)
            cb_ccw = md(p + 13 - s)

            @pl.when(s > 0)
            def _():
                pl.semaphore_wait(credit_r, 1)
                pl.semaphore_wait(credit_l, 1)

            rd_acw = pltpu.make_async_remote_copy(
                src_ref=sbuf_acw.at[slot], dst_ref=comm_acw.at[slot],
                send_sem=s1s.at[slot], recv_sem=s1r.at[slot],
                device_id=(right,), device_id_type=pl.DeviceIdType.MESH)
            rd_bcw = pltpu.make_async_remote_copy(
                src_ref=sbuf_bcw.at[slot], dst_ref=comm_bcw.at[slot],
                send_sem=s3s.at[slot], recv_sem=s3r.at[slot],
                device_id=(left,), device_id_type=pl.DeviceIdType.MESH)
            rd_acw.start()
            rd_bcw.start()

            @pl.when(s < H - 1)
            def _():
                rd_accw = pltpu.make_async_remote_copy(
                    src_ref=sbuf_accw.at[slot], dst_ref=comm_accw.at[slot],
                    send_sem=s2s.at[slot], recv_sem=s2r.at[slot],
                    device_id=(left,), device_id_type=pl.DeviceIdType.MESH)
                rd_bccw = pltpu.make_async_remote_copy(
                    src_ref=sbuf_bccw.at[slot], dst_ref=comm_bccw.at[slot],
                    send_sem=s4s.at[slot], recv_sem=s4r.at[slot],
                    device_id=(right,), device_id_type=pl.DeviceIdType.MESH)
                rd_accw.start()
                rd_bccw.start()

            gemm_a(ca_cw)
            gemm_b(cb_cw)

            @pl.when(s < H - 2)
            def _():
                gemm_a(ca_ccw)
                gemm_b(cb_ccw)

            rd_acw.wait()
            rd_bcw.wait()

            @pl.when(s < H - 1)
            def _():
                sum_a = out_ref[a_rows(ca_cw), :] + \
                    comm_acw[slot].astype(jnp.float32)
                sbuf_acw[pl.ds(nslot, 1)] = sum_a.astype(jnp.bfloat16)[None]
                sum_b = out_ref[b_rows(cb_cw), :] + \
                    comm_bcw[slot].astype(jnp.float32)
                sbuf_bcw[pl.ds(nslot, 1)] = sum_b.astype(jnp.bfloat16)[None]

            @pl.when(s < H - 1)
            def _():
                rd_accw = pltpu.make_async_remote_copy(
                    src_ref=sbuf_accw.at[slot], dst_ref=comm_accw.at[slot],
                    send_sem=s2s.at[slot], recv_sem=s2r.at[slot],
                    device_id=(left,), device_id_type=pl.DeviceIdType.MESH)
                rd_bccw = pltpu.make_async_remote_copy(
                    src_ref=sbuf_bccw.at[slot], dst_ref=comm_bccw.at[slot],
                    send_sem=s4s.at[slot], recv_sem=s4r.at[slot],
                    device_id=(right,), device_id_type=pl.DeviceIdType.MESH)
                rd_accw.wait()
                rd_bccw.wait()

            @pl.when(s < H - 2)
            def _():
                sum_a = out_ref[a_rows(ca_ccw), :] + \
                    comm_accw[slot].astype(jnp.float32)
                sbuf_accw[pl.ds(nslot, 1)] = sum_a.astype(jnp.bfloat16)[None]
                sum_b = out_ref[b_rows(cb_ccw), :] + \
                    comm_bccw[slot].astype(jnp.float32)
                sbuf_bccw[pl.ds(nslot, 1)] = sum_b.astype(jnp.bfloat16)[None]

            @pl.when(s < H - 2)
            def _():
                pl.semaphore_signal(credit_r, inc=1, device_id=(left,),
                                    device_id_type=pl.DeviceIdType.MESH)
                pl.semaphore_signal(credit_l, inc=1, device_id=(right,),
                                    device_id_type=pl.DeviceIdType.MESH)
            return carry

        lax.fori_loop(0, H, rs_step, 0)

        ra = md(p + 1)
        rb = md(p - 1)
        ya = jnp.maximum(
            out_ref[a_rows(ra), :]
            + comm_acw[1].astype(jnp.float32)
            + comm_accw[0].astype(jnp.float32), 0.0)
        yb = jnp.maximum(
            out_ref[b_rows(rb), :]
            + comm_bcw[1].astype(jnp.float32)
            + comm_bccw[0].astype(jnp.float32), 0.0)

        amax_local = jnp.maximum(jnp.max(ya), jnp.max(yb))
        amax_src[0, :] = jnp.full((128,), amax_local, jnp.float32)
        amax_buf[pl.ds(my, 1), :] = amax_src[0:1, :]

        def bcast_step(o, carry):
            tgt = md(my + o)
            rdma = pltpu.make_async_remote_copy(
                src_ref=amax_src.at[0],
                dst_ref=amax_buf.at[my],
                send_sem=amax_send.at[o],
                recv_sem=amax_recv.at[my],
                device_id=(tgt,),
                device_id_type=pl.DeviceIdType.MESH,
            )
            rdma.start()
            return carry

        lax.fori_loop(1, N_DEV, bcast_step, 0)

        def bcast_wait(o, carry):
            src_id = md(my + o)
            rdma = pltpu.make_async_remote_copy(
                src_ref=amax_src.at[0],
                dst_ref=amax_buf.at[src_id],
                send_sem=amax_send.at[o],
                recv_sem=amax_recv.at[src_id],
                device_id=(src_id,),
                device_id_type=pl.DeviceIdType.MESH,
            )
            rdma.wait_recv()
            return carry

        lax.fori_loop(1, N_DEV, bcast_wait, 0)

        amax = jnp.max(amax_buf[:, :])
        scale = amax / 127.0

        def q8(y):
            return jnp.clip(jnp.round(y / scale), -127.0, 127.0).astype(jnp.int8)

        q_full[a_rows(ra), :] = q8(ya)
        q_full[b_rows(rb), :] = q8(yb)

        def q_chunk_a(c):
            return q_full.at[a_rows(c), :]

        def q_chunk_b(c):
            return q_full.at[b_rows(c), :]

        def deq_a(c):
            out_ref[a_rows(c), :] = q_full[a_rows(c), :].astype(jnp.float32) * scale

        def deq_b(c):
            out_ref[b_rows(c), :] = q_full[b_rows(c), :].astype(jnp.float32) * scale

        def bcast_drain(o, carry):
            tgt = md(my + o)
            rdma = pltpu.make_async_remote_copy(
                src_ref=amax_src.at[0],
                dst_ref=amax_buf.at[my],
                send_sem=amax_send.at[o],
                recv_sem=amax_recv.at[my],
                device_id=(tgt,),
                device_id_type=pl.DeviceIdType.MESH,
            )
            rdma.wait_send()
            return carry

        lax.fori_loop(1, N_DEV, bcast_drain, 0)

        def ag_step(t, carry):
            slot = jnp.mod(t, 2)

            @pl.when(t > 0)
            def _():
                pl.semaphore_wait(credit_r2, 1)
                pl.semaphore_wait(credit_l2, 1)

            rd_ar = pltpu.make_async_remote_copy(
                src_ref=q_chunk_a(md(p + 1 - t)),
                dst_ref=q_chunk_a(md(p + 1 - t)),
                send_sem=s1s.at[slot], recv_sem=s1r.at[slot],
                device_id=(right,), device_id_type=pl.DeviceIdType.MESH)
            rd_bl = pltpu.make_async_remote_copy(
                src_ref=q_chunk_b(md(p - 1 + t)),
                dst_ref=q_chunk_b(md(p - 1 + t)),
                send_sem=s3s.at[slot], recv_sem=s3r.at[slot],
                device_id=(left,), device_id_type=pl.DeviceIdType.MESH)
            rd_ar.start()
            rd_bl.start()

            @pl.when(t < H - 1)
            def _():
                rd_al = pltpu.make_async_remote_copy(
                    src_ref=q_chunk_a(md(p + 1 + t)),
                    dst_ref=q_chunk_a(md(p + 1 + t)),
                    send_sem=s2s.at[slot], recv_sem=s2r.at[slot],
                    device_id=(left,), device_id_type=pl.DeviceIdType.MESH)
                rd_br = pltpu.make_async_remote_copy(
                    src_ref=q_chunk_b(md(p - 1 - t)),
                    dst_ref=q_chunk_b(md(p - 1 - t)),
                    send_sem=s4s.at[slot], recv_sem=s4r.at[slot],
                    device_id=(right,), device_id_type=pl.DeviceIdType.MESH)
                rd_al.start()
                rd_br.start()

            @pl.when(t == 0)
            def _():
                deq_a(ra)
                deq_b(rb)

            @pl.when(t > 0)
            def _():
                deq_a(md(p - t + 1))
                deq_a(md(p + t + 1))
                deq_b(md(p + t - 1))
                deq_b(md(p - t - 1))

            rd_ar.wait()
            rd_bl.wait()

            @pl.when(t < H - 1)
            def _():
                rd_al = pltpu.make_async_remote_copy(
                    src_ref=q_chunk_a(md(p + 1 + t)),
                    dst_ref=q_chunk_a(md(p + 1 + t)),
                    send_sem=s2s.at[slot], recv_sem=s2r.at[slot],
                    device_id=(left,), device_id_type=pl.DeviceIdType.MESH)
                rd_br = pltpu.make_async_remote_copy(
                    src_ref=q_chunk_b(md(p - 1 - t)),
                    dst_ref=q_chunk_b(md(p - 1 - t)),
                    send_sem=s4s.at[slot], recv_sem=s4r.at[slot],
                    device_id=(right,), device_id_type=pl.DeviceIdType.MESH)
                rd_al.wait()
                rd_br.wait()

            @pl.when(t < H - 2)
            def _():
                pl.semaphore_signal(credit_r2, inc=1, device_id=(left,),
                                    device_id_type=pl.DeviceIdType.MESH)
                pl.semaphore_signal(credit_l2, inc=1, device_id=(right,),
                                    device_id_type=pl.DeviceIdType.MESH)
            return carry

        lax.fori_loop(0, H, ag_step, 0)

        deq_a(md(p - 15))
        deq_b(md(p + 15))

    tables = jnp.array([_POS_OF_LOG, _RIGHT_OF_LOG, _LEFT_OF_LOG],
                       dtype=jnp.int32)

    bf = jnp.bfloat16
    return pl.pallas_call(
        body,
        out_shape=jax.ShapeDtypeStruct((m, n), jnp.float32),
        in_specs=[
            pl.BlockSpec(memory_space=pltpu.VMEM),
            pl.BlockSpec(memory_space=pltpu.VMEM),
            pl.BlockSpec(memory_space=pltpu.VMEM),
        ],
        out_specs=pl.BlockSpec(memory_space=pltpu.VMEM),
        scratch_shapes=[
            pltpu.VMEM((2, half // N_DEV, n), bf),
            pltpu.VMEM((2, half // N_DEV, n), bf),
            pltpu.VMEM((2, half // N_DEV, n), bf),
            pltpu.VMEM((2, half // N_DEV, n), bf),
            pltpu.VMEM((2, half // N_DEV, n), bf),
            pltpu.VMEM((2, half // N_DEV, n), bf),
            pltpu.VMEM((2, half // N_DEV, n), bf),
            pltpu.VMEM((2, half // N_DEV, n), bf),
            pltpu.SemaphoreType.DMA((2,)),
            pltpu.SemaphoreType.DMA((2,)),
            pltpu.SemaphoreType.DMA((2,)),
            pltpu.SemaphoreType.DMA((2,)),
            pltpu.SemaphoreType.DMA((2,)),
            pltpu.SemaphoreType.DMA((2,)),
            pltpu.SemaphoreType.DMA((2,)),
            pltpu.SemaphoreType.DMA((2,)),
            pltpu.SemaphoreType.REGULAR,
            pltpu.SemaphoreType.REGULAR,
            pltpu.SemaphoreType.REGULAR,
            pltpu.SemaphoreType.REGULAR,
            pltpu.VMEM((m, n), jnp.int8),
            pltpu.VMEM((1, 128), jnp.float32),
            pltpu.VMEM((N_DEV, 128), jnp.float32),
            pltpu.SemaphoreType.DMA((N_DEV,)),
            pltpu.SemaphoreType.DMA((N_DEV,)),
        ],
        compiler_params=pltpu.CompilerParams(
            collective_id=0,
            vmem_limit_bytes=56 * 1024 * 1024,
        ),
    )(tables, x, w_mat)
